# baseline (speedup 1.0000x reference)
import jax
import jax.numpy as jnp
from jax import lax
from jax.experimental import pallas as pl
from jax.experimental.pallas import tpu as pltpu
from jax.experimental.pallas import tpu_sc as plsc

N = 1024
D = 1024
E = 65536

NC = 2
NS = 16
HALF = N // NC
HW = HALF * N
SL = HW // NS
EPS = E // NS
GROUP = 128
NG = EPS // GROUP
ZCH = 2048
FIRE = 16


def _adj_body(ei_hbm, out_hbm, srcv, dstv, fidx, vals, zbuf, a_sh, zsem, esem, ssem):
    c = lax.axis_index("c")
    s = lax.axis_index("s")

    base = s * EPS
    eld = [
        pltpu.async_copy(ei_hbm.at[0, pl.ds(base, EPS)], srcv, esem),
        pltpu.async_copy(ei_hbm.at[1, pl.ds(base, EPS)], dstv, esem),
    ]

    def _fill_zero(i, carry):
        zbuf[pl.ds(i * 16, 16)] = jnp.zeros((16,), jnp.float32)
        return carry
    lax.fori_loop(0, ZCH // 16, _fill_zero, 0)

    zld = [
        pltpu.async_copy(zbuf, a_sh.at[pl.ds(s * SL + i * ZCH, ZCH)], zsem)
        for i in range(SL // ZCH)
    ]
    for dsc in eld:
        dsc.wait()

    def _grp(g, carry):
        for k in range(GROUP // 16):
            off = g * GROUP + k * 16
            sv = srcv[pl.ds(off, 16)]
            dv = dstv[pl.ds(off, 16)]
            mine = lax.shift_right_logical(dv, 9) == c
            loc = ((dv - c * HALF) * N + sv) & (HW - 1)
            fidx[g, pl.ds(k * 16, 16)] = loc
            vals[g, pl.ds(k * 16, 16)] = jnp.where(mine, 1.0, 0.0)
        return carry
    lax.fori_loop(0, NG, _grp, 0)

    for dsc in zld:
        dsc.wait()
    plsc.subcore_barrier()

    for lo in range(0, NG, FIRE):
        descs = [
            pltpu.async_copy(vals.at[g], a_sh.at[fidx.at[g]], ssem, add=True)
            for g in range(lo, lo + FIRE)
        ]
        for dsc in descs:
            dsc.wait()

    plsc.subcore_barrier()

    row0 = c * HALF + s * (SL // N)
    rdescs = [
        pltpu.async_copy(a_sh.at[pl.ds(s * SL + r * N, N)], out_hbm.at[row0 + r], zsem)
        for r in range(SL // N)
    ]
    for dsc in rdescs:
        dsc.wait()


_adj_kernel = pl.kernel(
    _adj_body,
    out_type=jax.ShapeDtypeStruct((N, N), jnp.float32),
    mesh=plsc.VectorSubcoreMesh(
        core_axis_name="c", subcore_axis_name="s", num_cores=NC, num_subcores=NS
    ),
    scratch_types=[
        pltpu.VMEM((EPS,), jnp.int32),
        pltpu.VMEM((EPS,), jnp.int32),
        pltpu.VMEM((NG, GROUP), jnp.int32),
        pltpu.VMEM((NG, GROUP), jnp.float32),
        pltpu.VMEM((ZCH,), jnp.float32),
        pltpu.VMEM_SHARED((HW,), jnp.float32),
        pltpu.SemaphoreType.DMA,
        pltpu.SemaphoreType.DMA,
        pltpu.SemaphoreType.DMA,
    ],
)


def _sigmoid(t):
    return 1.0 / (1.0 + jnp.exp(-t))


def _tc1_body(x_ref, gw_ref, whh_ref, bhh_ref, xw_ref, gh_ref):
    x = x_ref[...]
    xw_ref[...] = jnp.dot(x, gw_ref[...], preferred_element_type=jnp.float32)
    gh_ref[...] = lax.dot_general(x, whh_ref[...], (((1,), (1,)), ((), ())),
                                  preferred_element_type=jnp.float32) \
        + bhh_ref[...][None, :]


def _tc2_body(a_ref, x_ref, xw_ref, gh_ref, gb_ref, wih_ref, bih_ref,
              l1w_ref, l1b_ref, l2w_ref, l2b_ref, q_ref):
    acnt = a_ref[...]
    deg = jnp.sum(acnt, axis=1) + 1.0
    dinv = lax.rsqrt(deg)

    x = x_ref[...]
    xs = xw_ref[...] * dinv[:, None]
    agg = (jnp.dot(acnt, xs, preferred_element_type=jnp.float32) + xs) * dinv[:, None]
    nf = jnp.maximum(agg + gb_ref[...][None, :], 0.0)

    gi = lax.dot_general(nf, wih_ref[...], (((1,), (1,)), ((), ())),
                         preferred_element_type=jnp.float32) + bih_ref[...][None, :]
    gh = gh_ref[...]
    r = _sigmoid(gi[:, :D] + gh[:, :D])
    z = _sigmoid(gi[:, D:2 * D] + gh[:, D:2 * D])
    nt = jnp.tanh(gi[:, 2 * D:] + r * gh[:, 2 * D:])
    h = (1.0 - z) * nt + z * x

    srow = jnp.sum(h, axis=1)[None, :]
    h1 = jnp.maximum(
        lax.dot_general(srow, l1w_ref[...], (((1,), (1,)), ((), ())),
                        preferred_element_type=jnp.float32)
        + l1b_ref[...][None, :], 0.0)
    q_ref[...] = jnp.full((1, 1), jnp.sum(h1 * l2w_ref[...]) + l2b_ref[0],
                          jnp.float32)


def kernel(x, edge_index, edge_attr, gcn_W, gcn_b, W_ih, b_ih, W_hh, b_hh,
           lin1_W, lin1_b, lin2_W, lin2_b):
    del edge_attr
    a2 = _adj_kernel(edge_index)
    xw, gh = pl.pallas_call(
        _tc1_body,
        out_shape=(jax.ShapeDtypeStruct((N, D), jnp.float32),
                   jax.ShapeDtypeStruct((N, 3 * D), jnp.float32)),
    )(x, gcn_W, W_hh, b_hh)
    q = pl.pallas_call(
        _tc2_body,
        out_shape=jax.ShapeDtypeStruct((1, 1), jnp.float32),
    )(a2, x, xw, gh, gcn_b, W_ih, b_ih, lin1_W, lin1_b, lin2_W, lin2_b)
    return q.reshape(1)


# speedup vs baseline: 1.0246x; 1.0246x over previous
import jax
import jax.numpy as jnp
from jax import lax
from jax.experimental import pallas as pl
from jax.experimental.pallas import tpu as pltpu
from jax.experimental.pallas import tpu_sc as plsc

N = 1024
D = 1024
E = 65536

NC = 2
NS = 16
HALF = N // NC
HW = HALF * N
SL = HW // NS
EPS = E // NS
GROUP = 128
NG = EPS // GROUP
ZCH = 2048
FIRE = 16


def _adj_body(ei_hbm, out_hbm, srcv, dstv, fidx, vals, zbuf, a_sh, zsem, esem, ssem):
    c = lax.axis_index("c")
    s = lax.axis_index("s")

    base = s * EPS
    eld = [
        pltpu.async_copy(ei_hbm.at[0, pl.ds(base, EPS)], srcv, esem),
        pltpu.async_copy(ei_hbm.at[1, pl.ds(base, EPS)], dstv, esem),
    ]

    def _fill_zero(i, carry):
        zbuf[pl.ds(i * 16, 16)] = jnp.zeros((16,), jnp.float32)
        return carry
    lax.fori_loop(0, ZCH // 16, _fill_zero, 0)

    zld = [
        pltpu.async_copy(zbuf, a_sh.at[pl.ds(s * SL + i * ZCH, ZCH)], zsem)
        for i in range(SL // ZCH)
    ]
    for dsc in eld:
        dsc.wait()

    def _grp(g, carry):
        for k in range(GROUP // 16):
            off = g * GROUP + k * 16
            sv = srcv[pl.ds(off, 16)]
            dv = dstv[pl.ds(off, 16)]
            mine = lax.shift_right_logical(dv, 9) == c
            loc = ((dv - c * HALF) * N + sv) & (HW - 1)
            fidx[g, pl.ds(k * 16, 16)] = loc
            vals[g, pl.ds(k * 16, 16)] = jnp.where(mine, 1.0, 0.0)
        return carry
    lax.fori_loop(0, NG, _grp, 0)

    for dsc in zld:
        dsc.wait()
    plsc.subcore_barrier()

    for lo in range(0, NG, FIRE):
        descs = [
            pltpu.async_copy(vals.at[g], a_sh.at[fidx.at[g]], ssem, add=True)
            for g in range(lo, lo + FIRE)
        ]
        for dsc in descs:
            dsc.wait()

    plsc.subcore_barrier()

    row0 = c * HALF + s * (SL // N)
    rdescs = [
        pltpu.async_copy(a_sh.at[pl.ds(s * SL + r * N, N)], out_hbm.at[row0 + r], zsem)
        for r in range(SL // N)
    ]
    for dsc in rdescs:
        dsc.wait()


_adj_kernel = pl.kernel(
    _adj_body,
    out_type=jax.ShapeDtypeStruct((N, N), jnp.float32),
    mesh=plsc.VectorSubcoreMesh(
        core_axis_name="c", subcore_axis_name="s", num_cores=NC, num_subcores=NS
    ),
    scratch_types=[
        pltpu.VMEM((EPS,), jnp.int32),
        pltpu.VMEM((EPS,), jnp.int32),
        pltpu.VMEM((NG, GROUP), jnp.int32),
        pltpu.VMEM((NG, GROUP), jnp.float32),
        pltpu.VMEM((ZCH,), jnp.float32),
        pltpu.VMEM_SHARED((HW,), jnp.float32),
        pltpu.SemaphoreType.DMA,
        pltpu.SemaphoreType.DMA,
        pltpu.SemaphoreType.DMA,
    ],
)


def _sigmoid(t):
    return 1.0 / (1.0 + jnp.exp(-t))


def _tc1_body(x_ref, whh_ref, bhh_ref, gh_ref):
    gh_ref[...] = lax.dot_general(x_ref[...], whh_ref[...],
                                  (((1,), (1,)), ((), ())),
                                  preferred_element_type=jnp.float32) \
        + bhh_ref[...][None, :]


def _tc2_body(a_ref, x_ref, gh_ref, gw_ref, gb_ref, wih_ref, bih_ref,
              l1w_ref, l1b_ref, l2w_ref, l2b_ref, q_ref):
    acnt = a_ref[...]
    deg = jnp.sum(acnt, axis=1) + 1.0
    dinv = lax.rsqrt(deg)

    x = x_ref[...]
    xw = jnp.dot(x, gw_ref[...], preferred_element_type=jnp.float32)
    xs = xw * dinv[:, None]
    agg = (jnp.dot(acnt, xs, preferred_element_type=jnp.float32) + xs) * dinv[:, None]
    nf = jnp.maximum(agg + gb_ref[...][None, :], 0.0)

    gi = lax.dot_general(nf, wih_ref[...], (((1,), (1,)), ((), ())),
                         preferred_element_type=jnp.float32) + bih_ref[...][None, :]
    gh = gh_ref[...]
    r = _sigmoid(gi[:, :D] + gh[:, :D])
    z = _sigmoid(gi[:, D:2 * D] + gh[:, D:2 * D])
    nt = jnp.tanh(gi[:, 2 * D:] + r * gh[:, 2 * D:])
    h = (1.0 - z) * nt + z * x

    srow = jnp.sum(h, axis=1)[None, :]
    h1 = jnp.maximum(
        lax.dot_general(srow, l1w_ref[...], (((1,), (1,)), ((), ())),
                        preferred_element_type=jnp.float32)
        + l1b_ref[...][None, :], 0.0)
    q_ref[...] = jnp.full((1, 1), jnp.sum(h1 * l2w_ref[...]) + l2b_ref[0],
                          jnp.float32)


def kernel(x, edge_index, edge_attr, gcn_W, gcn_b, W_ih, b_ih, W_hh, b_hh,
           lin1_W, lin1_b, lin2_W, lin2_b):
    del edge_attr
    a2 = _adj_kernel(edge_index)
    gh = pl.pallas_call(
        _tc1_body,
        out_shape=jax.ShapeDtypeStruct((N, 3 * D), jnp.float32),
    )(x, W_hh, b_hh)
    q = pl.pallas_call(
        _tc2_body,
        out_shape=jax.ShapeDtypeStruct((1, 1), jnp.float32),
    )(a2, x, gh, gcn_W, gcn_b, W_ih, b_ih, lin1_W, lin1_b, lin2_W, lin2_b)
    return q.reshape(1)


# speedup vs baseline: 1.0417x; 1.0167x over previous
import jax
import jax.numpy as jnp
from jax import lax
from jax.experimental import pallas as pl
from jax.experimental.pallas import tpu as pltpu
from jax.experimental.pallas import tpu_sc as plsc

N = 1024
D = 1024
E = 65536

NC = 2
NS = 16
HALF = N // NC
HW = HALF * N
SL = HW // NS
EPS = E // NS
GROUP = 128
NG = EPS // GROUP
ZCH = 2048
FIRE = 16


def _adj_body(ei_hbm, out_hbm, srcv, dstv, fidx, vals, zbuf, a_sh, zsem, esem, ssem):
    c = lax.axis_index("c")
    s = lax.axis_index("s")

    base = s * EPS
    eld = [
        pltpu.async_copy(ei_hbm.at[0, pl.ds(base, EPS)], srcv, esem),
        pltpu.async_copy(ei_hbm.at[1, pl.ds(base, EPS)], dstv, esem),
    ]

    def _fill_zero(i, carry):
        zbuf[pl.ds(i * 16, 16)] = jnp.zeros((16,), jnp.float32)
        return carry
    lax.fori_loop(0, ZCH // 16, _fill_zero, 0)

    zld = [
        pltpu.async_copy(zbuf, a_sh.at[pl.ds(s * SL + i * ZCH, ZCH)], zsem)
        for i in range(SL // ZCH)
    ]
    for dsc in eld:
        dsc.wait()

    def _grp(g, carry):
        for k in range(GROUP // 16):
            off = g * GROUP + k * 16
            sv = srcv[pl.ds(off, 16)]
            dv = dstv[pl.ds(off, 16)]
            mine = lax.shift_right_logical(dv, 9) == c
            loc = ((dv - c * HALF) * N + sv) & (HW - 1)
            fidx[g, pl.ds(k * 16, 16)] = loc
            vals[g, pl.ds(k * 16, 16)] = jnp.where(mine, 1.0, 0.0)
        return carry
    lax.fori_loop(0, NG, _grp, 0)

    for dsc in zld:
        dsc.wait()
    plsc.subcore_barrier()

    for lo in range(0, NG, FIRE):
        descs = [
            pltpu.async_copy(vals.at[g], a_sh.at[fidx.at[g]], ssem, add=True)
            for g in range(lo, lo + FIRE)
        ]
        for dsc in descs:
            dsc.wait()

    plsc.subcore_barrier()

    row0 = c * HALF + s * (SL // N)
    rdescs = [
        pltpu.async_copy(a_sh.at[pl.ds(s * SL + r * N, N)], out_hbm.at[row0 + r], zsem)
        for r in range(SL // N)
    ]
    for dsc in rdescs:
        dsc.wait()


_adj_kernel = pl.kernel(
    _adj_body,
    out_type=jax.ShapeDtypeStruct((N, N), jnp.float32),
    mesh=plsc.VectorSubcoreMesh(
        core_axis_name="c", subcore_axis_name="s", num_cores=NC, num_subcores=NS
    ),
    scratch_types=[
        pltpu.VMEM((EPS,), jnp.int32),
        pltpu.VMEM((EPS,), jnp.int32),
        pltpu.VMEM((NG, GROUP), jnp.int32),
        pltpu.VMEM((NG, GROUP), jnp.float32),
        pltpu.VMEM((ZCH,), jnp.float32),
        pltpu.VMEM_SHARED((HW,), jnp.float32),
        pltpu.SemaphoreType.DMA,
        pltpu.SemaphoreType.DMA,
        pltpu.SemaphoreType.DMA,
    ],
)


def _sigmoid(t):
    return 1.0 / (1.0 + jnp.exp(-t))


TB = 256


def _tc1_body(x_ref, whh_ref, bhh_ref, gw_ref, gh_ref, xw_ref):
    x = x_ref[...]
    gh_ref[...] = lax.dot_general(x, whh_ref[...], (((1,), (1,)), ((), ())),
                                  preferred_element_type=jnp.float32) \
        + bhh_ref[...][None, :]
    xw_ref[...] = jnp.dot(x, gw_ref[...], preferred_element_type=jnp.float32)


def _tc2_body(a_ref, x_ref, gh_ref, xw_ref, gb_ref, wih_ref, bih_ref,
              l1w_ref, l1b_ref, l2w_ref, l2b_ref, q_ref):
    acnt = a_ref[...]
    deg = jnp.sum(acnt, axis=1) + 1.0
    dinv = lax.rsqrt(deg)

    x = x_ref[...]
    xs = xw_ref[...] * dinv[:, None]
    agg = (jnp.dot(acnt, xs, preferred_element_type=jnp.float32) + xs) * dinv[:, None]
    nf = jnp.maximum(agg + gb_ref[...][None, :], 0.0)

    gi = lax.dot_general(nf, wih_ref[...], (((1,), (1,)), ((), ())),
                         preferred_element_type=jnp.float32) + bih_ref[...][None, :]
    gh = gh_ref[...]
    r = _sigmoid(gi[:, :D] + gh[:, :D])
    z = _sigmoid(gi[:, D:2 * D] + gh[:, D:2 * D])
    nt = jnp.tanh(gi[:, 2 * D:] + r * gh[:, 2 * D:])
    h = (1.0 - z) * nt + z * x

    srow = jnp.sum(h, axis=1)[None, :]
    h1 = jnp.maximum(
        lax.dot_general(srow, l1w_ref[...], (((1,), (1,)), ((), ())),
                        preferred_element_type=jnp.float32)
        + l1b_ref[...][None, :], 0.0)
    q_ref[...] = jnp.full((1, 1), jnp.sum(h1 * l2w_ref[...]) + l2b_ref[0],
                          jnp.float32)


def kernel(x, edge_index, edge_attr, gcn_W, gcn_b, W_ih, b_ih, W_hh, b_hh,
           lin1_W, lin1_b, lin2_W, lin2_b):
    del edge_attr
    a2 = _adj_kernel(edge_index)
    gh, xw = pl.pallas_call(
        _tc1_body,
        grid=(N // TB,),
        in_specs=[
            pl.BlockSpec((TB, D), lambda i: (i, 0)),
            pl.BlockSpec((3 * D, D), lambda i: (0, 0)),
            pl.BlockSpec((3 * D,), lambda i: (0,)),
            pl.BlockSpec((D, D), lambda i: (0, 0)),
        ],
        out_specs=[
            pl.BlockSpec((TB, 3 * D), lambda i: (i, 0)),
            pl.BlockSpec((TB, D), lambda i: (i, 0)),
        ],
        out_shape=(jax.ShapeDtypeStruct((N, 3 * D), jnp.float32),
                   jax.ShapeDtypeStruct((N, D), jnp.float32)),
    )(x, W_hh, b_hh, gcn_W)
    q = pl.pallas_call(
        _tc2_body,
        out_shape=jax.ShapeDtypeStruct((1, 1), jnp.float32),
    )(a2, x, gh, xw, gcn_b, W_ih, b_ih, lin1_W, lin1_b, lin2_W, lin2_b)
    return q.reshape(1)


# speedup vs baseline: 1.0484x; 1.0065x over previous
import jax
import jax.numpy as jnp
from jax import lax
from jax.experimental import pallas as pl
from jax.experimental.pallas import tpu as pltpu
from jax.experimental.pallas import tpu_sc as plsc

N = 1024
D = 1024
E = 65536

NC = 2
NS = 16
HALF = N // NC
HW = HALF * N
SL = HW // NS
EPS = E // NS
GROUP = 128
NG = EPS // GROUP
ZCH = 2048
FIRE = 16


def _adj_body(ei_hbm, out_hbm, srcv, dstv, fidx, vals, zbuf, a_sh, zsem, esem, ssem):
    c = lax.axis_index("c")
    s = lax.axis_index("s")

    base = s * EPS
    eld = [
        pltpu.async_copy(ei_hbm.at[0, pl.ds(base, EPS)], srcv, esem),
        pltpu.async_copy(ei_hbm.at[1, pl.ds(base, EPS)], dstv, esem),
    ]

    def _fill_zero(i, carry):
        zbuf[pl.ds(i * 16, 16)] = jnp.zeros((16,), jnp.float32)
        return carry
    lax.fori_loop(0, ZCH // 16, _fill_zero, 0)

    zld = [
        pltpu.async_copy(zbuf, a_sh.at[pl.ds(s * SL + i * ZCH, ZCH)], zsem)
        for i in range(SL // ZCH)
    ]
    for dsc in eld:
        dsc.wait()

    def _grp(g, carry):
        for k in range(GROUP // 16):
            off = g * GROUP + k * 16
            sv = srcv[pl.ds(off, 16)]
            dv = dstv[pl.ds(off, 16)]
            mine = lax.shift_right_logical(dv, 9) == c
            loc = ((dv - c * HALF) * N + sv) & (HW - 1)
            fidx[g, pl.ds(k * 16, 16)] = loc
            vals[g, pl.ds(k * 16, 16)] = jnp.where(mine, 1.0, 0.0)
        return carry
    lax.fori_loop(0, NG, _grp, 0)

    for dsc in zld:
        dsc.wait()
    plsc.subcore_barrier()

    for lo in range(0, NG, FIRE):
        descs = [
            pltpu.async_copy(vals.at[g], a_sh.at[fidx.at[g]], ssem, add=True)
            for g in range(lo, lo + FIRE)
        ]
        for dsc in descs:
            dsc.wait()

    plsc.subcore_barrier()

    row0 = c * HALF + s * (SL // N)
    rdescs = [
        pltpu.async_copy(a_sh.at[pl.ds(s * SL + r * N, N)], out_hbm.at[row0 + r], zsem)
        for r in range(SL // N)
    ]
    for dsc in rdescs:
        dsc.wait()


_adj_kernel = pl.kernel(
    _adj_body,
    out_type=jax.ShapeDtypeStruct((N, N), jnp.float32),
    mesh=plsc.VectorSubcoreMesh(
        core_axis_name="c", subcore_axis_name="s", num_cores=NC, num_subcores=NS
    ),
    scratch_types=[
        pltpu.VMEM((EPS,), jnp.int32),
        pltpu.VMEM((EPS,), jnp.int32),
        pltpu.VMEM((NG, GROUP), jnp.int32),
        pltpu.VMEM((NG, GROUP), jnp.float32),
        pltpu.VMEM((ZCH,), jnp.float32),
        pltpu.VMEM_SHARED((HW,), jnp.float32),
        pltpu.SemaphoreType.DMA,
        pltpu.SemaphoreType.DMA,
        pltpu.SemaphoreType.DMA,
    ],
)


def _sigmoid(t):
    return 0.5 * (jnp.tanh(0.5 * t) + 1.0)


TB = 256


def _tc1_body(x_ref, whh_ref, bhh_ref, gw_ref, gh_ref, xw_ref):
    x = x_ref[...]
    gh_ref[...] = lax.dot_general(x, whh_ref[...], (((1,), (1,)), ((), ())),
                                  preferred_element_type=jnp.float32) \
        + bhh_ref[...][None, :]
    xw_ref[...] = jnp.dot(x, gw_ref[...], preferred_element_type=jnp.float32)


def _tc2_body(a_ref, x_ref, gh_ref, xw_ref, gb_ref, wih_ref, bih_ref,
              l1w_ref, l1b_ref, l2w_ref, l2b_ref, q_ref):
    acnt = a_ref[...]
    deg = jnp.sum(acnt, axis=1) + 1.0
    dinv = lax.rsqrt(deg)

    x = x_ref[...]
    xs = xw_ref[...] * dinv[:, None]
    agg = (jnp.dot(acnt, xs, preferred_element_type=jnp.float32) + xs) * dinv[:, None]
    nf = jnp.maximum(agg + gb_ref[...][None, :], 0.0)

    gi = lax.dot_general(nf, wih_ref[...], (((1,), (1,)), ((), ())),
                         preferred_element_type=jnp.float32) + bih_ref[...][None, :]
    gh = gh_ref[...]
    r = _sigmoid(gi[:, :D] + gh[:, :D])
    z = _sigmoid(gi[:, D:2 * D] + gh[:, D:2 * D])
    nt = jnp.tanh(gi[:, 2 * D:] + r * gh[:, 2 * D:])
    h = nt + z * (x - nt)

    srow = jnp.sum(h, axis=1)[None, :]
    h1 = jnp.maximum(
        lax.dot_general(srow, l1w_ref[...], (((1,), (1,)), ((), ())),
                        preferred_element_type=jnp.float32)
        + l1b_ref[...][None, :], 0.0)
    q_ref[...] = jnp.full((1, 1), jnp.sum(h1 * l2w_ref[...]) + l2b_ref[0],
                          jnp.float32)


def kernel(x, edge_index, edge_attr, gcn_W, gcn_b, W_ih, b_ih, W_hh, b_hh,
           lin1_W, lin1_b, lin2_W, lin2_b):
    del edge_attr
    a2 = _adj_kernel(edge_index)
    gh, xw = pl.pallas_call(
        _tc1_body,
        grid=(N // TB,),
        in_specs=[
            pl.BlockSpec((TB, D), lambda i: (i, 0)),
            pl.BlockSpec((3 * D, D), lambda i: (0, 0)),
            pl.BlockSpec((3 * D,), lambda i: (0,)),
            pl.BlockSpec((D, D), lambda i: (0, 0)),
        ],
        out_specs=[
            pl.BlockSpec((TB, 3 * D), lambda i: (i, 0)),
            pl.BlockSpec((TB, D), lambda i: (i, 0)),
        ],
        out_shape=(jax.ShapeDtypeStruct((N, 3 * D), jnp.float32),
                   jax.ShapeDtypeStruct((N, D), jnp.float32)),
    )(x, W_hh, b_hh, gcn_W)
    q = pl.pallas_call(
        _tc2_body,
        out_shape=jax.ShapeDtypeStruct((1, 1), jnp.float32),
    )(a2, x, gh, xw, gcn_b, W_ih, b_ih, lin1_W, lin1_b, lin2_W, lin2_b)
    return q.reshape(1)
